# Initial kernel scaffold; baseline (speedup 1.0000x reference)
import jax
import jax.numpy as jnp
from jax.experimental import pallas as pl


def kernel(graph, feat, efeat, W_self1, W_nbr1, b1, W_self2, W_nbr2, b2):
    raise NotImplementedError("write your pallas kernel here")


# speedup vs baseline: 1.9703x; 1.9703x over previous
import functools

import jax
import jax.numpy as jnp
from jax import lax
from jax.experimental import pallas as pl
from jax.experimental.pallas import tpu as pltpu
from jax.experimental.pallas import tpu_sc as plsc

_N = 10000
_E = 320000
_D = 128

_NC = 2
_NS = 16
_NW = _NC * _NS
_EPW = _E // _NW
_CH = 80
_NCHUNK = _EPW // _CH
_RPT = 624
_RPT_LAST = _N - 15 * _RPT
_ZR = 16


def _sc_agg_body(feat_hbm, src_hbm, dst_hbm, efeat_hbm, out_hbm,
                 agg_sh, srcv, dstv, rows, ef, zbuf, gsem):
    c = lax.axis_index("c")
    s = lax.axis_index("s")
    wid = s * _NC + c

    zero = jnp.zeros((16,), jnp.float32)

    def _zb(i, _):
        zbuf[i // 8, pl.ds((i % 8) * 16, 16)] = zero
        return ()

    lax.fori_loop(0, _ZR * 8, _zb, ())
    row0 = s * _RPT
    nz = jnp.where(s == _NS - 1, _RPT_LAST // _ZR, _RPT // _ZR)

    def _z(i, _):
        off = pl.multiple_of(row0 + i * _ZR, 8)
        pltpu.sync_copy(zbuf, agg_sh.at[pl.ds(off, _ZR), :])
        return ()

    lax.fori_loop(0, nz, _z, ())
    plsc.subcore_barrier()

    ebase = wid * _EPW

    def _chunk(i, _):
        base = pl.multiple_of(ebase + i * _CH, 8)
        pltpu.sync_copy(src_hbm.at[pl.ds(base, _CH)], srcv)
        pltpu.sync_copy(dst_hbm.at[pl.ds(base, _CH)], dstv)
        pltpu.async_copy(feat_hbm.at[srcv], rows, gsem).wait()
        pltpu.sync_copy(efeat_hbm.at[pl.ds(base, _CH), :], ef)

        def _ew(j, _):
            e = j // 8
            k = (j % 8) * 16
            x = ef[e, pl.ds(k, 16)]
            g = 1.0 / (1.0 + jnp.exp(-x))
            rows[e, pl.ds(k, 16)] = rows[e, pl.ds(k, 16)] * g
            return ()

        lax.fori_loop(0, _CH * 8, _ew, ())
        pltpu.sync_copy(rows, agg_sh.at[dstv], add=True)
        return ()

    lax.fori_loop(0, _NCHUNK, _chunk, ())

    plsc.subcore_barrier()

    @pl.when(s < _NS - 1)
    def _():
        off = pl.multiple_of(row0, 8)
        pltpu.sync_copy(agg_sh.at[pl.ds(off, _RPT), :],
                        out_hbm.at[c, pl.ds(off, _RPT), :])

    @pl.when(s == _NS - 1)
    def _():
        off = 15 * _RPT
        pltpu.sync_copy(agg_sh.at[pl.ds(off, _RPT_LAST), :],
                        out_hbm.at[c, pl.ds(off, _RPT_LAST), :])


_sc_agg = functools.partial(
    pl.kernel,
    out_type=jax.ShapeDtypeStruct((_NC, _N, _D), jnp.float32),
    mesh=plsc.VectorSubcoreMesh(core_axis_name="c", subcore_axis_name="s"),
    scratch_types=[
        pltpu.VMEM_SHARED((_N, _D), jnp.float32),
        pltpu.VMEM((_CH,), jnp.int32),
        pltpu.VMEM((_CH,), jnp.int32),
        pltpu.VMEM((_CH, _D), jnp.float32),
        pltpu.VMEM((_CH, _D), jnp.float32),
        pltpu.VMEM((_ZR, _D), jnp.float32),
        pltpu.SemaphoreType.DMA,
    ],
)(_sc_agg_body)


_RB = 1000


def _dense_body(feat_ref, p0_ref, p1_ref, ws_ref, wn_ref, b_ref, out_ref):
    agg = p0_ref[...] + p1_ref[...]
    h = jnp.dot(feat_ref[...], ws_ref[...], preferred_element_type=jnp.float32)
    h += jnp.dot(agg, wn_ref[...], preferred_element_type=jnp.float32)
    h += b_ref[...]
    out_ref[...] = jnp.maximum(h, 0.0) + feat_ref[...]


def _dense(feat, p0, p1, w_self, w_nbr, b):
    grid = (_N // _RB,)
    blk = pl.BlockSpec((_RB, _D), lambda i: (i, 0))
    wblk = pl.BlockSpec((_D, _D), lambda i: (0, 0))
    bblk = pl.BlockSpec((1, _D), lambda i: (0, 0))
    return pl.pallas_call(
        _dense_body,
        grid=grid,
        in_specs=[blk, blk, blk, wblk, wblk, bblk],
        out_specs=blk,
        out_shape=jax.ShapeDtypeStruct((_N, _D), jnp.float32),
    )(feat, p0, p1, w_self, w_nbr, b.reshape(1, _D))


@jax.jit
def kernel(graph, feat, efeat, W_self1, W_nbr1, b1, W_self2, W_nbr2, b2):
    src = graph[0]
    dst = graph[1]
    p1 = _sc_agg(feat, src, dst, efeat)
    h1 = _dense(feat, p1[0], p1[1], W_self1, W_nbr1, b1)
    p2 = _sc_agg(h1, src, dst, efeat)
    h2 = _dense(h1, p2[0], p2[1], W_self2, W_nbr2, b2)
    return h2
